# baseline (speedup 1.0000x reference)
import functools

import jax
import jax.numpy as jnp
from jax import lax
from jax.experimental import pallas as pl
from jax.experimental.pallas import tpu as pltpu
from jax.experimental.pallas import tpu_sc as plsc

N = 10000
E = 320000
D = 128
DH = D // 2
C = 40
BN_EPS = 1e-5

NC = 2
NS = 16
EW = E // NS
CH = 128
NFULL = EW // CH
TAIL = EW - NFULL * CH
NCHN = N // CH
NTAIL = N - NCHN * CH
NB = 4
CW = 8
GRP = NFULL // NB


def _agg_body(with_counts, dh, gsub, xa_hbm, xb_hbm, src_hbm, dst_hbm,
              zr_hbm, *rest):
    GR = CH * gsub
    NGRP = NFULL // gsub
    if with_counts:
        (on_hbm, out_a, out_b, out_cnt, src_v, dst2_v, dstt2_v, rows_v, rowst_v,
         ones_v, zbuf_v, zbuf16_v, *sems) = rest
        gsems = sems[:NB]
        ssems = sems[NB:2 * NB]
        csem, isem, acc_sh, cnt_sh = sems[2 * NB:]
    else:
        (out_a, out_b, src_v, dst2_v, dstt2_v, rows_v, rowst_v,
         zbuf_v, *sems) = rest
        out_cnt = ones_v = zbuf16_v = csem = cnt_sh = None
        gsems = sems[:NB]
        ssems = sems[NB:2 * NB]
        isem, acc_sh = sems[2 * NB:]

    cid = lax.axis_index("c")
    sid = lax.axis_index("s")

    pltpu.sync_copy(zr_hbm, zbuf_v)
    if with_counts:
        pltpu.sync_copy(on_hbm.at[0], ones_v)
        pltpu.sync_copy(on_hbm.at[1], zbuf16_v)

    ebase = sid * EW
    pltpu.async_copy(src_hbm.at[pl.ds(ebase, EW)], src_v, isem)
    def iload(j, carry):
        pltpu.async_copy(dst_hbm.at[pl.ds(ebase + j * CH, CH)], dst2_v.at[j],
                         isem)
        return carry
    lax.fori_loop(0, NFULL, iload, 0)
    if TAIL:
        pltpu.async_copy(dst_hbm.at[pl.ds(ebase + NFULL * CH, TAIL)],
                         dstt2_v.at[0], isem)

    def zchunk(j, carry):
        @pl.when(j % NS == sid)
        def _():
            pltpu.sync_copy(zbuf_v, acc_sh.at[pl.ds(j * CH, CH)])
            if with_counts:
                pltpu.sync_copy(zbuf16_v, cnt_sh.at[pl.ds(j * CH, CH)])
        return carry
    lax.fori_loop(0, NCHN, zchunk, 0)
    if NTAIL:
        @pl.when(sid == NCHN % NS)
        def _ztail():
            off = NCHN * CH
            pltpu.sync_copy(zbuf_v.at[pl.ds(0, NTAIL)], acc_sh.at[pl.ds(off, NTAIL)])
            if with_counts:
                pltpu.sync_copy(zbuf16_v.at[pl.ds(0, NTAIL)],
                                cnt_sh.at[pl.ds(off, NTAIL)])
    plsc.subcore_barrier()

    pltpu.make_async_copy(dst_hbm.at[pl.ds(0, EW)], src_v, isem).wait()
    pltpu.make_async_copy(dst_hbm.at[pl.ds(0, EW)], src_v, isem).wait()

    def run(x_hbm, do_counts):
        def slot(b, i):
            pltpu.make_async_copy(x_hbm.at[pl.ds(0, GR)], rows_v.at[b],
                                  gsems[b]).wait()
            for s2 in range(gsub):
                ci = i * gsub + s2
                pltpu.async_copy(rows_v.at[b, pl.ds(s2 * CH, CH)],
                                 acc_sh.at[dst2_v.at[ci]], ssems[b], add=True)
                if do_counts:
                    @pl.when(ci % NC == cid)
                    def _():
                        pltpu.async_copy(ones_v, cnt_sh.at[dst2_v.at[ci]],
                                         csem, add=True)

            @pl.when(i + NB < NGRP)
            def _():
                pltpu.make_async_copy(x_hbm.at[pl.ds(0, GR)], rows_v.at[b],
                                      ssems[b]).wait()
                pltpu.async_copy(
                    x_hbm.at[src_v.at[pl.ds((i + NB) * GR, GR)]],
                    rows_v.at[b], gsems[b])

        for b in range(NB):
            pltpu.async_copy(x_hbm.at[src_v.at[pl.ds(b * GR, GR)]],
                             rows_v.at[b], gsems[b])

        def grp(g, carry):
            for b in range(NB):
                slot(b, g * NB + b)
            return carry
        lax.fori_loop(0, NGRP // NB, grp, 0)
        for i in range(NGRP - NGRP % NB, NGRP):
            slot(i % NB, i)
        for b in range(NB):
            pltpu.make_async_copy(x_hbm.at[pl.ds(0, GR)], rows_v.at[b],
                                  ssems[b]).wait()
        if do_counts:
            nr = NFULL * CH * CW // dh // NC
            pltpu.make_async_copy(x_hbm.at[pl.ds(0, nr)],
                                  acc_sh.at[pl.ds(0, nr)], csem).wait()
        if TAIL:
            toff = NFULL * CH
            pltpu.async_copy(x_hbm.at[src_v.at[pl.ds(toff, TAIL)]], rowst_v,
                             gsems[0]).wait()
            pltpu.sync_copy(rowst_v, acc_sh.at[dstt2_v.at[0]], add=True)
            if do_counts:
                @pl.when(cid == 0)
                def _():
                    pltpu.sync_copy(ones_v.at[pl.ds(0, TAIL)],
                                    cnt_sh.at[dstt2_v.at[0]], add=True)

    @pl.when(cid == 0)
    def _core0():
        run(xa_hbm, with_counts)

    @pl.when(cid == 1)
    def _core1():
        run(xb_hbm, with_counts)

    plsc.subcore_barrier()

    def wchunk(j, carry):
        @pl.when(j % NS == sid)
        def _():
            @pl.when(cid == 0)
            def _a():
                pltpu.sync_copy(acc_sh.at[pl.ds(j * CH, CH)],
                                out_a.at[pl.ds(j * CH, CH)])
            @pl.when(cid == 1)
            def _b():
                pltpu.sync_copy(acc_sh.at[pl.ds(j * CH, CH)],
                                out_b.at[pl.ds(j * CH, CH)])
            if with_counts:
                pltpu.sync_copy(cnt_sh.at[pl.ds(j * CH, CH)],
                                out_cnt.at[cid, pl.ds(j * CH, CH)])
        return carry
    lax.fori_loop(0, NCHN, wchunk, 0)
    if NTAIL:
        @pl.when(sid == NCHN % NS)
        def _wtail():
            off = NCHN * CH
            @pl.when(cid == 0)
            def _a():
                pltpu.sync_copy(acc_sh.at[pl.ds(off, NTAIL)],
                                out_a.at[pl.ds(off, NTAIL)])
            @pl.when(cid == 1)
            def _b():
                pltpu.sync_copy(acc_sh.at[pl.ds(off, NTAIL)],
                                out_b.at[pl.ds(off, NTAIL)])
            if with_counts:
                pltpu.sync_copy(cnt_sh.at[pl.ds(off, NTAIL)],
                                out_cnt.at[cid, pl.ds(off, NTAIL)])


def _make_agg(with_counts, dh=DH, gsub=2):
    out_type = [jax.ShapeDtypeStruct((N, dh), jnp.float32),
                jax.ShapeDtypeStruct((N, dh), jnp.float32)]
    if with_counts:
        out_type.append(jax.ShapeDtypeStruct((NC, N, CW), jnp.float32))
    return pl.kernel(
        functools.partial(_agg_body, with_counts, dh, gsub),
        out_type=tuple(out_type),
        mesh=plsc.VectorSubcoreMesh(core_axis_name="c", subcore_axis_name="s",
                                    num_cores=NC, num_subcores=NS),
        scratch_types=(
            [
                pltpu.VMEM((EW,), jnp.int32),
                pltpu.VMEM((NFULL, CH), jnp.int32),
                pltpu.VMEM((1, TAIL), jnp.int32),
                pltpu.VMEM((NB, CH * gsub, dh), jnp.float32),
                pltpu.VMEM((TAIL, dh), jnp.float32),
            ]
            + ([pltpu.VMEM((CH, CW), jnp.float32)] if with_counts else [])
            + [pltpu.VMEM((CH, dh), jnp.float32)]
            + ([pltpu.VMEM((CH, CW), jnp.float32)] if with_counts else [])
            + [pltpu.SemaphoreType.DMA] * (2 * NB)
            + ([pltpu.SemaphoreType.DMA] if with_counts else [])
            + [
                pltpu.SemaphoreType.DMA,
                pltpu.VMEM_SHARED((N, dh), jnp.float32),
            ]
            + ([pltpu.VMEM_SHARED((N, CW), jnp.float32)] if with_counts else [])
        ),
        compiler_params=pltpu.CompilerParams(use_tc_tiling_on_sc=False),
        name="gin_agg",
    )


_agg_with_counts = _make_agg(True, gsub=1)
_agg_no_counts = _make_agg(False, gsub=1)
SQ = 32
_agg_small = _make_agg(False, dh=SQ, gsub=1)

BN = 1000
CQ = C // 2
SQP = 32

_half_spec = pl.BlockSpec((BN, DH), lambda i: (i, 0))
_q_spec = pl.BlockSpec((BN, SQP), lambda i: (i, 0))
_c_spec = pl.BlockSpec((BN, C), lambda i: (i, 0))
_cnt_spec = pl.BlockSpec((NC, BN, CW), lambda i: (0, i, 0))
_w_spec = pl.BlockSpec((D, D), lambda i: (0, 0))
_wc_spec = pl.BlockSpec((D, C), lambda i: (0, 0))
_b_spec = pl.BlockSpec((1, D), lambda i: (0, 0))
_bc_spec = pl.BlockSpec((1, C), lambda i: (0, 0))


def _l1_body(x_ref, sa_ref, sb_ref, c_ref, w0_ref, b0_ref, w1_ref,
             oa_ref, ob_ref):
    inv = 1.0 / jnp.maximum(c_ref[0, :, 0:1] + c_ref[1, :, 0:1], 1.0)
    m = jnp.concatenate([sa_ref[...], sb_ref[...]], axis=1) * inv
    h1 = jnp.maximum(
        jnp.dot(x_ref[...] + m, w0_ref[...], preferred_element_type=jnp.float32)
        + b0_ref[...], 0.0)
    z = jnp.dot(h1, w1_ref[...], preferred_element_type=jnp.float32)
    oa_ref[...] = z[:, :DH]
    ob_ref[...] = z[:, DH:]


def _mid_body(ya_ref, yb_ref, sa_ref, sb_ref, c_ref, b_ref, w_ref,
              oa_ref, ob_ref):
    inv = 1.0 / jnp.maximum(c_ref[0, :, 0:1] + c_ref[1, :, 0:1], 1.0)
    y = jnp.concatenate([ya_ref[...], yb_ref[...]], axis=1)
    m = jnp.concatenate([sa_ref[...], sb_ref[...]], axis=1) * inv
    hn = jnp.maximum(y + m + b_ref[...], 0.0)
    z = jnp.dot(hn, w_ref[...], preferred_element_type=jnp.float32)
    oa_ref[...] = z[:, :DH]
    ob_ref[...] = z[:, DH:]


def _pre3_body(ya_ref, yb_ref, sa_ref, sb_ref, c_ref, b_ref, w2_ref, wp_ref,
               oa_ref, ob_ref, op_ref):
    inv = 1.0 / jnp.maximum(c_ref[0, :, 0:1] + c_ref[1, :, 0:1], 1.0)
    y = jnp.concatenate([ya_ref[...], yb_ref[...]], axis=1)
    m = jnp.concatenate([sa_ref[...], sb_ref[...]], axis=1) * inv
    hn = jnp.maximum(y + m + b_ref[...], 0.0)
    z = jnp.dot(hn, w2_ref[...], preferred_element_type=jnp.float32)
    zpad = jnp.zeros((BN, SQP - CQ), jnp.float32)
    oa_ref[...] = jnp.concatenate([z[:, :CQ], zpad], axis=1)
    ob_ref[...] = jnp.concatenate([z[:, CQ:], zpad], axis=1)
    op_ref[...] = jnp.dot(hn, wp_ref[...], preferred_element_type=jnp.float32)


def _fin_body(ya_ref, yb_ref, sa_ref, sb_ref, c_ref, b_ref, yp_ref, bp_ref,
              o_ref):
    inv = 1.0 / jnp.maximum(c_ref[0, :, 0:1] + c_ref[1, :, 0:1], 1.0)
    y = jnp.concatenate([ya_ref[:, :CQ], yb_ref[:, :CQ]], axis=1)
    m = jnp.concatenate([sa_ref[:, :CQ], sb_ref[:, :CQ]], axis=1) * inv
    h3 = jnp.maximum(y + m + b_ref[...], 0.0)
    o_ref[...] = (yp_ref[...] + bp_ref[...] + h3) * 0.5


def _tc(body, in_specs, out_specs, out_shape):
    return pl.pallas_call(body, grid=(N // BN,), in_specs=in_specs,
                          out_specs=out_specs, out_shape=out_shape)


_half_out = [jax.ShapeDtypeStruct((N, DH), jnp.float32)] * 2
_q_out = [jax.ShapeDtypeStruct((N, SQP), jnp.float32)] * 2


def kernel(h, edge_index, W0, b0, g0, be0, W1, b1, g1, be1, W2, b2, g2, be2, Wp, bp):
    src = edge_index[0]
    dst = edge_index[1]
    s = 1.0 / jnp.sqrt(jnp.float32(1.0 + BN_EPS))

    w0_eff = W0.T * (g0 * s)[None, :]
    b0_eff = (b0 * g0 * s + be0).reshape(1, D)
    w1_eff = W1.T * (g1 * s)[None, :]
    b1_eff = (b1 * g1 * s + be1).reshape(1, D)
    w2_eff = W2.T * (g2 * s)[None, :]
    b2_eff = (b2 * g2 * s + be2).reshape(1, C)
    wp_t = Wp.T
    bp2 = bp.reshape(1, C)

    zr = jnp.zeros((CH, DH), jnp.float32)
    zrq = jnp.zeros((CH, SQP), jnp.float32)
    on = jnp.zeros((2, CH, CW), jnp.float32).at[0, :, 0].set(1.0)

    ha, hb = h[:, :DH], h[:, DH:]
    s0a, s0b, cnt = _agg_with_counts(ha, hb, src, dst, zr, on)
    y1a, y1b = _tc(_l1_body,
                   [pl.BlockSpec((BN, D), lambda i: (i, 0)), _half_spec,
                    _half_spec, _cnt_spec, _w_spec, _b_spec, _w_spec],
                   [_half_spec, _half_spec], _half_out)(
                       h, s0a, s0b, cnt, w0_eff, b0_eff, w1_eff)
    s1a, s1b = _agg_no_counts(y1a, y1b, src, dst, zr)
    y2a, y2b, yp = _tc(_pre3_body,
                       [_half_spec] * 4 + [_cnt_spec, _b_spec, _wc_spec,
                                           _wc_spec],
                       [_q_spec, _q_spec, _c_spec],
                       _q_out + [jax.ShapeDtypeStruct((N, C), jnp.float32)])(
                           y1a, y1b, s1a, s1b, cnt, b1_eff, w2_eff, wp_t)
    s2a, s2b = _agg_small(y2a, y2b, src, dst, zrq)
    return _tc(_fin_body,
               [_q_spec] * 4 + [_cnt_spec, _bc_spec, _c_spec, _bc_spec],
               _c_spec, jax.ShapeDtypeStruct((N, C), jnp.float32))(
                   y2a, y2b, s2a, s2b, cnt, b2_eff, yp, bp2)


# speedup vs baseline: 1.0058x; 1.0058x over previous
import functools

import jax
import jax.numpy as jnp
from jax import lax
from jax.experimental import pallas as pl
from jax.experimental.pallas import tpu as pltpu
from jax.experimental.pallas import tpu_sc as plsc

N = 10000
E = 320000
D = 128
DH = D // 2
C = 40
BN_EPS = 1e-5

NC = 2
NS = 16
EW = E // NS
CH = 128
NFULL = EW // CH
TAIL = EW - NFULL * CH
NCHN = N // CH
NTAIL = N - NCHN * CH
NB = 4
CW = 8
GRP = NFULL // NB


def _agg_body(with_counts, dh, gsub, xa_hbm, xb_hbm, src_hbm, dst_hbm,
              zr_hbm, *rest):
    GR = CH * gsub
    NGRP = NFULL // gsub
    if with_counts:
        (on_hbm, out_a, out_b, out_cnt, src_v, dst2_v, dstt2_v, rows_v, rowst_v,
         ones_v, zbuf_v, zbuf16_v, *sems) = rest
        gsems = sems[:NB]
        ssems = sems[NB:2 * NB]
        csem, isem, acc_sh, cnt_sh = sems[2 * NB:]
    else:
        (out_a, out_b, src_v, dst2_v, dstt2_v, rows_v, rowst_v,
         zbuf_v, *sems) = rest
        out_cnt = ones_v = zbuf16_v = csem = cnt_sh = None
        gsems = sems[:NB]
        ssems = sems[NB:2 * NB]
        isem, acc_sh = sems[2 * NB:]

    cid = lax.axis_index("c")
    sid = lax.axis_index("s")

    pltpu.sync_copy(zr_hbm, zbuf_v)
    if with_counts:
        pltpu.sync_copy(on_hbm.at[0], ones_v)
        pltpu.sync_copy(on_hbm.at[1], zbuf16_v)

    ebase = sid * EW
    pltpu.async_copy(src_hbm.at[pl.ds(ebase, EW)], src_v, isem)
    def iload(j, carry):
        pltpu.async_copy(dst_hbm.at[pl.ds(ebase + j * CH, CH)], dst2_v.at[j],
                         isem)
        return carry
    lax.fori_loop(0, NFULL, iload, 0)
    if TAIL:
        pltpu.async_copy(dst_hbm.at[pl.ds(ebase + NFULL * CH, TAIL)],
                         dstt2_v.at[0], isem)

    def zchunk(j, carry):
        @pl.when(j % NS == sid)
        def _():
            pltpu.sync_copy(zbuf_v, acc_sh.at[pl.ds(j * CH, CH)])
            if with_counts:
                pltpu.sync_copy(zbuf16_v, cnt_sh.at[pl.ds(j * CH, CH)])
        return carry
    lax.fori_loop(0, NCHN, zchunk, 0)
    if NTAIL:
        @pl.when(sid == NCHN % NS)
        def _ztail():
            off = NCHN * CH
            pltpu.sync_copy(zbuf_v.at[pl.ds(0, NTAIL)], acc_sh.at[pl.ds(off, NTAIL)])
            if with_counts:
                pltpu.sync_copy(zbuf16_v.at[pl.ds(0, NTAIL)],
                                cnt_sh.at[pl.ds(off, NTAIL)])
    plsc.subcore_barrier()

    pltpu.make_async_copy(dst_hbm.at[pl.ds(0, EW)], src_v, isem).wait()
    pltpu.make_async_copy(dst_hbm.at[pl.ds(0, EW)], src_v, isem).wait()

    def run(x_hbm, do_counts):
        def slot(b, i):
            pltpu.make_async_copy(x_hbm.at[pl.ds(0, GR)], rows_v.at[b],
                                  gsems[b]).wait()
            for s2 in range(gsub):
                ci = i * gsub + s2
                pltpu.async_copy(rows_v.at[b, pl.ds(s2 * CH, CH)],
                                 acc_sh.at[dst2_v.at[ci]], ssems[b], add=True)
                if do_counts:
                    pltpu.async_copy(ones_v, cnt_sh.at[dst2_v.at[ci]], csem,
                                     add=True)

            @pl.when(i + NB < NGRP)
            def _():
                pltpu.make_async_copy(x_hbm.at[pl.ds(0, GR)], rows_v.at[b],
                                      ssems[b]).wait()
                pltpu.async_copy(
                    x_hbm.at[src_v.at[pl.ds((i + NB) * GR, GR)]],
                    rows_v.at[b], gsems[b])

        for b in range(NB):
            pltpu.async_copy(x_hbm.at[src_v.at[pl.ds(b * GR, GR)]],
                             rows_v.at[b], gsems[b])

        def grp(g, carry):
            for b in range(NB):
                slot(b, g * NB + b)
            return carry
        lax.fori_loop(0, NGRP // NB, grp, 0)
        for i in range(NGRP - NGRP % NB, NGRP):
            slot(i % NB, i)
        for b in range(NB):
            pltpu.make_async_copy(x_hbm.at[pl.ds(0, GR)], rows_v.at[b],
                                  ssems[b]).wait()
        if do_counts:
            pltpu.make_async_copy(x_hbm.at[pl.ds(0, NFULL * CH * CW // dh)],
                                  acc_sh.at[pl.ds(0, NFULL * CH * CW // dh)],
                                  csem).wait()
        if TAIL:
            toff = NFULL * CH
            pltpu.async_copy(x_hbm.at[src_v.at[pl.ds(toff, TAIL)]], rowst_v,
                             gsems[0]).wait()
            pltpu.sync_copy(rowst_v, acc_sh.at[dstt2_v.at[0]], add=True)
            if do_counts:
                pltpu.sync_copy(ones_v.at[pl.ds(0, TAIL)],
                                cnt_sh.at[dstt2_v.at[0]], add=True)

    @pl.when(cid == 0)
    def _core0():
        run(xa_hbm, with_counts)

    @pl.when(cid == 1)
    def _core1():
        run(xb_hbm, False)

    plsc.subcore_barrier()

    def wchunk(j, carry):
        @pl.when(j % NS == sid)
        def _():
            @pl.when(cid == 0)
            def _a():
                pltpu.sync_copy(acc_sh.at[pl.ds(j * CH, CH)],
                                out_a.at[pl.ds(j * CH, CH)])
                if with_counts:
                    pltpu.sync_copy(cnt_sh.at[pl.ds(j * CH, CH)],
                                    out_cnt.at[pl.ds(j * CH, CH)])
            @pl.when(cid == 1)
            def _b():
                pltpu.sync_copy(acc_sh.at[pl.ds(j * CH, CH)],
                                out_b.at[pl.ds(j * CH, CH)])
        return carry
    lax.fori_loop(0, NCHN, wchunk, 0)
    if NTAIL:
        @pl.when(sid == NCHN % NS)
        def _wtail():
            off = NCHN * CH
            @pl.when(cid == 0)
            def _a():
                pltpu.sync_copy(acc_sh.at[pl.ds(off, NTAIL)],
                                out_a.at[pl.ds(off, NTAIL)])
                if with_counts:
                    pltpu.sync_copy(cnt_sh.at[pl.ds(off, NTAIL)],
                                    out_cnt.at[pl.ds(off, NTAIL)])
            @pl.when(cid == 1)
            def _b():
                pltpu.sync_copy(acc_sh.at[pl.ds(off, NTAIL)],
                                out_b.at[pl.ds(off, NTAIL)])


def _make_agg(with_counts, dh=DH, gsub=2):
    out_type = [jax.ShapeDtypeStruct((N, dh), jnp.float32),
                jax.ShapeDtypeStruct((N, dh), jnp.float32)]
    if with_counts:
        out_type.append(jax.ShapeDtypeStruct((N, CW), jnp.float32))
    return pl.kernel(
        functools.partial(_agg_body, with_counts, dh, gsub),
        out_type=tuple(out_type),
        mesh=plsc.VectorSubcoreMesh(core_axis_name="c", subcore_axis_name="s",
                                    num_cores=NC, num_subcores=NS),
        scratch_types=(
            [
                pltpu.VMEM((EW,), jnp.int32),
                pltpu.VMEM((NFULL, CH), jnp.int32),
                pltpu.VMEM((1, TAIL), jnp.int32),
                pltpu.VMEM((NB, CH * gsub, dh), jnp.float32),
                pltpu.VMEM((TAIL, dh), jnp.float32),
            ]
            + ([pltpu.VMEM((CH, CW), jnp.float32)] if with_counts else [])
            + [pltpu.VMEM((CH, dh), jnp.float32)]
            + ([pltpu.VMEM((CH, CW), jnp.float32)] if with_counts else [])
            + [pltpu.SemaphoreType.DMA] * (2 * NB)
            + ([pltpu.SemaphoreType.DMA] if with_counts else [])
            + [
                pltpu.SemaphoreType.DMA,
                pltpu.VMEM_SHARED((N, dh), jnp.float32),
            ]
            + ([pltpu.VMEM_SHARED((N, CW), jnp.float32)] if with_counts else [])
        ),
        compiler_params=pltpu.CompilerParams(use_tc_tiling_on_sc=False),
        name="gin_agg",
    )


_agg_with_counts = _make_agg(True, gsub=1)
_agg_no_counts = _make_agg(False, gsub=1)
SQ = 32
_agg_small = _make_agg(False, dh=SQ, gsub=1)

BN = 1000
CQ = C // 2
SQP = 32

_half_spec = pl.BlockSpec((BN, DH), lambda i: (i, 0))
_q_spec = pl.BlockSpec((BN, SQP), lambda i: (i, 0))
_c_spec = pl.BlockSpec((BN, C), lambda i: (i, 0))
_cnt_spec = pl.BlockSpec((BN, CW), lambda i: (i, 0))
_w_spec = pl.BlockSpec((D, D), lambda i: (0, 0))
_wc_spec = pl.BlockSpec((D, C), lambda i: (0, 0))
_b_spec = pl.BlockSpec((1, D), lambda i: (0, 0))
_bc_spec = pl.BlockSpec((1, C), lambda i: (0, 0))


def _l1_body(x_ref, sa_ref, sb_ref, c_ref, w0_ref, b0_ref, w1_ref,
             oa_ref, ob_ref):
    inv = 1.0 / jnp.maximum(c_ref[:, 0:1], 1.0)
    m = jnp.concatenate([sa_ref[...], sb_ref[...]], axis=1) * inv
    h1 = jnp.maximum(
        jnp.dot(x_ref[...] + m, w0_ref[...], preferred_element_type=jnp.float32)
        + b0_ref[...], 0.0)
    z = jnp.dot(h1, w1_ref[...], preferred_element_type=jnp.float32)
    oa_ref[...] = z[:, :DH]
    ob_ref[...] = z[:, DH:]


def _mid_body(ya_ref, yb_ref, sa_ref, sb_ref, c_ref, b_ref, w_ref,
              oa_ref, ob_ref):
    inv = 1.0 / jnp.maximum(c_ref[:, 0:1], 1.0)
    y = jnp.concatenate([ya_ref[...], yb_ref[...]], axis=1)
    m = jnp.concatenate([sa_ref[...], sb_ref[...]], axis=1) * inv
    hn = jnp.maximum(y + m + b_ref[...], 0.0)
    z = jnp.dot(hn, w_ref[...], preferred_element_type=jnp.float32)
    oa_ref[...] = z[:, :DH]
    ob_ref[...] = z[:, DH:]


def _pre3_body(ya_ref, yb_ref, sa_ref, sb_ref, c_ref, b_ref, w2_ref, wp_ref,
               oa_ref, ob_ref, op_ref):
    inv = 1.0 / jnp.maximum(c_ref[:, 0:1], 1.0)
    y = jnp.concatenate([ya_ref[...], yb_ref[...]], axis=1)
    m = jnp.concatenate([sa_ref[...], sb_ref[...]], axis=1) * inv
    hn = jnp.maximum(y + m + b_ref[...], 0.0)
    z = jnp.dot(hn, w2_ref[...], preferred_element_type=jnp.float32)
    zpad = jnp.zeros((BN, SQP - CQ), jnp.float32)
    oa_ref[...] = jnp.concatenate([z[:, :CQ], zpad], axis=1)
    ob_ref[...] = jnp.concatenate([z[:, CQ:], zpad], axis=1)
    op_ref[...] = jnp.dot(hn, wp_ref[...], preferred_element_type=jnp.float32)


def _fin_body(ya_ref, yb_ref, sa_ref, sb_ref, c_ref, b_ref, yp_ref, bp_ref,
              o_ref):
    inv = 1.0 / jnp.maximum(c_ref[:, 0:1], 1.0)
    y = jnp.concatenate([ya_ref[:, :CQ], yb_ref[:, :CQ]], axis=1)
    m = jnp.concatenate([sa_ref[:, :CQ], sb_ref[:, :CQ]], axis=1) * inv
    h3 = jnp.maximum(y + m + b_ref[...], 0.0)
    o_ref[...] = (yp_ref[...] + bp_ref[...] + h3) * 0.5


def _tc(body, in_specs, out_specs, out_shape):
    return pl.pallas_call(body, grid=(N // BN,), in_specs=in_specs,
                          out_specs=out_specs, out_shape=out_shape)


_half_out = [jax.ShapeDtypeStruct((N, DH), jnp.float32)] * 2
_q_out = [jax.ShapeDtypeStruct((N, SQP), jnp.float32)] * 2


def kernel(h, edge_index, W0, b0, g0, be0, W1, b1, g1, be1, W2, b2, g2, be2, Wp, bp):
    src = edge_index[0]
    dst = edge_index[1]
    s = 1.0 / jnp.sqrt(jnp.float32(1.0 + BN_EPS))

    w0_eff = W0.T * (g0 * s)[None, :]
    b0_eff = (b0 * g0 * s + be0).reshape(1, D)
    w1_eff = W1.T * (g1 * s)[None, :]
    b1_eff = (b1 * g1 * s + be1).reshape(1, D)
    w2_eff = W2.T * (g2 * s)[None, :]
    b2_eff = (b2 * g2 * s + be2).reshape(1, C)
    wp_t = Wp.T
    bp2 = bp.reshape(1, C)

    zr = jnp.zeros((CH, DH), jnp.float32)
    zrq = jnp.zeros((CH, SQP), jnp.float32)
    on = jnp.zeros((2, CH, CW), jnp.float32).at[0, :, 0].set(1.0)

    ha, hb = h[:, :DH], h[:, DH:]
    s0a, s0b, cnt = _agg_with_counts(ha, hb, src, dst, zr, on)
    y1a, y1b = _tc(_l1_body,
                   [pl.BlockSpec((BN, D), lambda i: (i, 0)), _half_spec,
                    _half_spec, _cnt_spec, _w_spec, _b_spec, _w_spec],
                   [_half_spec, _half_spec], _half_out)(
                       h, s0a, s0b, cnt, w0_eff, b0_eff, w1_eff)
    s1a, s1b = _agg_no_counts(y1a, y1b, src, dst, zr)
    y2a, y2b, yp = _tc(_pre3_body,
                       [_half_spec] * 4 + [_cnt_spec, _b_spec, _wc_spec,
                                           _wc_spec],
                       [_q_spec, _q_spec, _c_spec],
                       _q_out + [jax.ShapeDtypeStruct((N, C), jnp.float32)])(
                           y1a, y1b, s1a, s1b, cnt, b1_eff, w2_eff, wp_t)
    s2a, s2b = _agg_small(y2a, y2b, src, dst, zrq)
    return _tc(_fin_body,
               [_q_spec] * 4 + [_cnt_spec, _bc_spec, _c_spec, _bc_spec],
               _c_spec, jax.ShapeDtypeStruct((N, C), jnp.float32))(
                   y2a, y2b, s2a, s2b, cnt, b2_eff, yp, bp2)
